# Initial kernel scaffold; baseline (speedup 1.0000x reference)
import jax
import jax.numpy as jnp
from jax.experimental import pallas as pl


def kernel(indices, codebook):
    raise NotImplementedError("write your pallas kernel here")


# speedup vs baseline: 97.1476x; 97.1476x over previous
import functools

import jax
import jax.numpy as jnp
from jax import lax
from jax.experimental import pallas as pl
from jax.experimental.pallas import tpu as pltpu
from jax.experimental.pallas import tpu_sc as plsc

_LANES = 16
_NC, _NS = 2, 16
_NW = _NC * _NS

_T = 256


@functools.cache
def _build(N, K, B, D):
    rows_per_w = N // _NW
    steps = rows_per_w // _T
    idx_per_step = _T * K
    g_rows = 128
    n_gath = idx_per_step // g_rows

    mesh = plsc.VectorSubcoreMesh(core_axis_name="c", subcore_axis_name="s")

    @functools.partial(
        pl.kernel,
        mesh=mesh,
        out_type=jax.ShapeDtypeStruct((N * K, D), jnp.float32),
        compiler_params=pltpu.CompilerParams(use_tc_tiling_on_sc=False),
        scratch_types=[
            pltpu.VMEM((idx_per_step,), jnp.int32),
            pltpu.VMEM((idx_per_step, D), jnp.float32),
            pltpu.SemaphoreType.DMA,
        ],
    )
    def pq(idx_hbm, cb_hbm, out_hbm, idx_v, rows_v, sem):
        wid = lax.axis_index("s") * _NC + lax.axis_index("c")
        offs = lax.iota(jnp.int32, _LANES) * B

        def step(t, carry):
            base = (wid * rows_per_w + t * _T) * K
            pltpu.sync_copy(idx_hbm.at[pl.ds(base, idx_per_step)], idx_v)

            def add_offs(c, carry):
                sl = pl.ds(c * _LANES, _LANES)
                idx_v[sl] = idx_v[sl] + offs
                return carry

            lax.fori_loop(0, idx_per_step // _LANES, add_offs, 0)

            def fire(g, carry):
                sl = pl.ds(g * g_rows, g_rows)
                pltpu.async_copy(cb_hbm.at[idx_v.at[sl]], rows_v.at[sl], sem)
                return carry

            lax.fori_loop(0, n_gath, fire, 0)

            def drain(g, carry):
                sl = pl.ds(g * g_rows, g_rows)
                pltpu.make_async_copy(
                    cb_hbm.at[idx_v.at[sl]], rows_v.at[sl], sem).wait()
                return carry

            lax.fori_loop(0, n_gath, drain, 0)

            pltpu.sync_copy(rows_v, out_hbm.at[pl.ds(base, idx_per_step)])
            return carry

        lax.fori_loop(0, steps, step, 0)

    return pq


def kernel(indices, codebook):
    N, K = indices.shape
    _, B, D = codebook.shape
    pq = _build(N, K, B, D)
    out = pq(indices.reshape(-1), codebook.reshape(K * B, D))
    return out.reshape(N, K * D)


# speedup vs baseline: 98.1036x; 1.0098x over previous
import functools

import jax
import jax.numpy as jnp
from jax import lax
from jax.experimental import pallas as pl
from jax.experimental.pallas import tpu as pltpu
from jax.experimental.pallas import tpu_sc as plsc

_LANES = 16
_NC, _NS = 2, 16
_NW = _NC * _NS

_T = 256


@functools.cache
def _build(N, K, B, D):
    rows_per_w = N // _NW
    steps = rows_per_w // _T
    idx_per_step = _T * K

    mesh = plsc.VectorSubcoreMesh(core_axis_name="c", subcore_axis_name="s")

    @functools.partial(
        pl.kernel,
        mesh=mesh,
        out_type=jax.ShapeDtypeStruct((N * K, D), jnp.float32),
        compiler_params=pltpu.CompilerParams(use_tc_tiling_on_sc=False),
        scratch_types=[
            pltpu.VMEM((_T, K), jnp.int32),
            pltpu.VMEM((idx_per_step,), jnp.int32),
            pltpu.VMEM((idx_per_step, D), jnp.float32),
            pltpu.SemaphoreType.DMA,
        ],
    )
    def pq(idx_hbm, cb_hbm, out_hbm, idx2_v, idx_v, rows_v, sem):
        wid = lax.axis_index("s") * _NC + lax.axis_index("c")
        offs = lax.iota(jnp.int32, _LANES) * B

        def step(t, carry):
            row = wid * rows_per_w + t * _T
            pltpu.sync_copy(idx_hbm.at[pl.ds(row, _T), :], idx2_v)

            def add_offs(r, carry):
                idx_v[pl.ds(r * _LANES, _LANES)] = idx2_v[r, :] + offs
                return carry

            lax.fori_loop(0, _T, add_offs, 0)

            pltpu.async_copy(cb_hbm.at[idx_v], rows_v, sem).wait()
            pltpu.sync_copy(rows_v, out_hbm.at[pl.ds(row * K, idx_per_step)])
            return carry

        lax.fori_loop(0, steps, step, 0)

    return pq


def kernel(indices, codebook):
    N, K = indices.shape
    _, B, D = codebook.shape
    pq = _build(N, K, B, D)
    out = pq(indices, codebook.reshape(K * B, D))
    return out.reshape(N, K * D)


# speedup vs baseline: 117.6283x; 1.1990x over previous
import functools

import jax
import jax.numpy as jnp
from jax import lax
from jax.experimental import pallas as pl
from jax.experimental.pallas import tpu as pltpu
from jax.experimental.pallas import tpu_sc as plsc

_LANES = 16
_NC, _NS = 2, 16
_NW = _NC * _NS

_T = 256


@functools.cache
def _build(N, K, B, D):
    rows_per_w = N // _NW
    steps = rows_per_w // _T
    idx_per_step = _T * K
    assert steps % 2 == 0 and steps >= 4

    mesh = plsc.VectorSubcoreMesh(core_axis_name="c", subcore_axis_name="s")

    @functools.partial(
        pl.kernel,
        mesh=mesh,
        out_type=jax.ShapeDtypeStruct((N * K, D), jnp.float32),
        compiler_params=pltpu.CompilerParams(use_tc_tiling_on_sc=False),
        scratch_types=[
            pltpu.VMEM((_T, K), jnp.int32),
            pltpu.VMEM((_T, K), jnp.int32),
            pltpu.VMEM((idx_per_step,), jnp.int32),
            pltpu.VMEM((idx_per_step,), jnp.int32),
            pltpu.VMEM((idx_per_step, D), jnp.float32),
            pltpu.VMEM((idx_per_step, D), jnp.float32),
            pltpu.SemaphoreType.DMA,
            pltpu.SemaphoreType.DMA,
            pltpu.SemaphoreType.DMA,
            pltpu.SemaphoreType.DMA,
            pltpu.SemaphoreType.DMA,
            pltpu.SemaphoreType.DMA,
        ],
    )
    def pq(idx_hbm, cb_hbm, out_hbm,
           idx2_a, idx2_b, idx1_a, idx1_b, rows_a, rows_b,
           semi_a, semi_b, semg_a, semg_b, sems_a, sems_b):
        wid = lax.axis_index("s") * _NC + lax.axis_index("c")
        offs = lax.iota(jnp.int32, _LANES) * B
        idx2 = (idx2_a, idx2_b)
        idx1 = (idx1_a, idx1_b)
        rows = (rows_a, rows_b)
        semi = (semi_a, semi_b)
        semg = (semg_a, semg_b)
        sems = (sems_a, sems_b)

        def idx_copy(t, p):
            row = wid * rows_per_w + t * _T
            return pltpu.make_async_copy(
                idx_hbm.at[pl.ds(row, _T), :], idx2[p], semi[p])

        def gather_copy(t, p):
            return pltpu.make_async_copy(cb_hbm.at[idx1[p]], rows[p], semg[p])

        def store_copy(t, p):
            base = (wid * rows_per_w + t * _T) * K
            return pltpu.make_async_copy(
                rows[p], out_hbm.at[pl.ds(base, idx_per_step)], sems[p])

        def add_offs(p):
            def body(r, carry):
                idx1[p][pl.ds(r * _LANES, _LANES)] = idx2[p][r, :] + offs
                return carry
            lax.fori_loop(0, _T, body, 0)

        idx_copy(0, 0).start()
        idx_copy(1, 1).start()
        idx_copy(0, 0).wait()
        add_offs(0)

        def loop(t2, carry):
            for p in (0, 1):
                t = t2 * 2 + p
                @pl.when(t2 >= 1)
                def _():
                    store_copy(t - 2, p).wait()
                gather_copy(t, p).start()
                @pl.when(t + 2 < steps)
                def _():
                    idx_copy(t + 2, p).start()
                @pl.when(t + 1 < steps)
                def _():
                    idx_copy(t + 1, 1 - p).wait()
                    add_offs(1 - p)
                gather_copy(t, p).wait()
                store_copy(t, p).start()
            return carry

        lax.fori_loop(0, steps // 2, loop, 0)
        store_copy(steps - 2, 0).wait()
        store_copy(steps - 1, 1).wait()

    return pq


def kernel(indices, codebook):
    N, K = indices.shape
    _, B, D = codebook.shape
    pq = _build(N, K, B, D)
    out = pq(indices, codebook.reshape(K * B, D))
    return out.reshape(N, K * D)
